# baseline (speedup 1.0000x reference)
import functools

import jax
import jax.numpy as jnp
from jax import lax
from jax.experimental import pallas as pl
from jax.experimental.pallas import tpu as pltpu
from jax.experimental.pallas import tpu_sc as plsc

N = 6400000
G = 1024
NW = 32
ROWS = N // NW
S = 8000
NCH = ROWS // S
VPC = S // 16

_mesh = plsc.VectorSubcoreMesh(core_axis_name="c", subcore_axis_name="s")


@functools.partial(
    pl.kernel,
    mesh=_mesh,
    out_type=jax.ShapeDtypeStruct((NW, G), jnp.float32),
    compiler_params=pltpu.CompilerParams(needs_layout_passes=False),
    scratch_types=[
        pltpu.VMEM((3 * S,), jnp.float32),
        pltpu.VMEM((S,), jnp.float32),
        pltpu.VMEM((S,), jnp.int32),
        pltpu.VMEM((16, G), jnp.float32),
        pltpu.VMEM((G,), jnp.float32),
    ],
)
def _seg_kernel(pos_hbm, ch_hbm, b_hbm, out_hbm, pos_v, ch_v, b_v, acc_v, part_v):
    wid = lax.axis_index("s") * 2 + lax.axis_index("c")
    lane = lax.iota(jnp.int32, 16)
    zero16 = jnp.zeros((16,), jnp.float32)

    def zero_body(j, _):
        for l in range(16):
            acc_v[l, pl.ds(j * 16, 16)] = zero16
        return 0

    lax.fori_loop(0, G // 16, zero_body, 0)

    base0 = wid * ROWS

    def chunk_body(cidx, _):
        base = base0 + cidx * S
        pltpu.sync_copy(pos_hbm.at[pl.ds(base * 3, S * 3)], pos_v)
        pltpu.sync_copy(ch_hbm.at[pl.ds(base, S)], ch_v)
        pltpu.sync_copy(b_hbm.at[pl.ds(base, S)], b_v)

        def vec_body(i, _):
            idx3 = i * 48 + lane * 3
            p = plsc.load_gather(pos_v, [idx3])
            c = ch_v[pl.ds(i * 16, 16)]
            b = b_v[pl.ds(i * 16, 16)]
            plsc.addupdate_scatter(acc_v, [lane, b], p * c)
            return 0

        lax.fori_loop(0, VPC, vec_body, 0)
        return 0

    lax.fori_loop(0, NCH, chunk_body, 0)

    def red_body(j, _):
        s = acc_v[0, pl.ds(j * 16, 16)]
        for l in range(1, 16):
            s = s + acc_v[l, pl.ds(j * 16, 16)]
        part_v[pl.ds(j * 16, 16)] = s
        return 0

    lax.fori_loop(0, G // 16, red_body, 0)

    pltpu.sync_copy(part_v, out_hbm.at[wid])


def kernel(positions, node_charges, batch):
    pos_flat = positions.reshape(-1)
    ch = node_charges.reshape(-1)
    partials = _seg_kernel(pos_flat, ch, batch)
    return partials.sum(axis=0).reshape(G, 1)


# speedup vs baseline: 25.9737x; 25.9737x over previous
import functools

import jax
import jax.numpy as jnp
from jax import lax
from jax.experimental import pallas as pl
from jax.experimental.pallas import tpu as pltpu
from jax.experimental.pallas import tpu_sc as plsc

N = 6400000
G = 1024
NW = 32
ROWS = N // NW
S = 8000
NCH = ROWS // S
VPC = S // 16

_mesh = plsc.VectorSubcoreMesh(core_axis_name="c", subcore_axis_name="s")


@functools.partial(
    pl.kernel,
    mesh=_mesh,
    out_type=jax.ShapeDtypeStruct((NW, G), jnp.float32),
    compiler_params=pltpu.CompilerParams(needs_layout_passes=False),
    scratch_types=[
        pltpu.VMEM((S,), jnp.float32),
        pltpu.VMEM((S,), jnp.float32),
        pltpu.VMEM((S,), jnp.int32),
        pltpu.VMEM((16, G), jnp.float32),
        pltpu.VMEM((G,), jnp.float32),
    ],
)
def _seg_kernel(x_hbm, ch_hbm, b_hbm, out_hbm, x_v, ch_v, b_v, acc_v, part_v):
    wid = lax.axis_index("s") * 2 + lax.axis_index("c")
    lane = lax.iota(jnp.int32, 16)
    zero16f = jnp.zeros((16,), jnp.float32)

    def zero_body(j, _):
        for l in range(16):
            acc_v[l, pl.ds(j * 16, 16)] = zero16f
        return 0

    lax.fori_loop(0, G // 16, zero_body, 0)

    base0 = wid * ROWS

    def chunk_body(cidx, _):
        base = base0 + cidx * S
        pltpu.sync_copy(x_hbm.at[pl.ds(base, S)], x_v)
        pltpu.sync_copy(ch_hbm.at[pl.ds(base, S)], ch_v)
        pltpu.sync_copy(b_hbm.at[pl.ds(base, S)], b_v)

        def vec_body(i, _):
            p = x_v[pl.ds(i * 16, 16)]
            c = ch_v[pl.ds(i * 16, 16)]
            b = b_v[pl.ds(i * 16, 16)]
            plsc.addupdate_scatter(acc_v, [lane, b], p * c)
            return 0

        lax.fori_loop(0, VPC, vec_body, 0)
        return 0

    lax.fori_loop(0, NCH, chunk_body, 0)

    def red_body(j, _):
        s = acc_v[0, pl.ds(j * 16, 16)]
        for l in range(1, 16):
            s = s + acc_v[l, pl.ds(j * 16, 16)]
        part_v[pl.ds(j * 16, 16)] = s
        return 0

    lax.fori_loop(0, G // 16, red_body, 0)

    pltpu.sync_copy(part_v, out_hbm.at[wid])


def kernel(positions, node_charges, batch):
    x0 = positions[:, 0]
    c0 = node_charges[:, 0]
    partials = _seg_kernel(x0, c0, batch)
    return partials.sum(axis=0).reshape(G, 1)


# speedup vs baseline: 56.7804x; 2.1861x over previous
import functools

import jax
import jax.numpy as jnp
from jax import lax
from jax.experimental import pallas as pl
from jax.experimental.pallas import tpu as pltpu
from jax.experimental.pallas import tpu_sc as plsc

N = 6400000
G = 1024
NW = 32
ROWS = N // NW
S = 10000
NCH = ROWS // S
HALF = NCH // 2
VPC = S // 16

_mesh = plsc.VectorSubcoreMesh(core_axis_name="c", subcore_axis_name="s")


@functools.partial(
    pl.kernel,
    mesh=_mesh,
    out_type=jax.ShapeDtypeStruct((NW, G), jnp.float32),
    compiler_params=pltpu.CompilerParams(needs_layout_passes=False),
    scratch_types=[
        pltpu.VMEM((S,), jnp.float32),
        pltpu.VMEM((S,), jnp.float32),
        pltpu.VMEM((S,), jnp.int32),
        pltpu.VMEM((S,), jnp.float32),
        pltpu.VMEM((S,), jnp.float32),
        pltpu.VMEM((S,), jnp.int32),
        pltpu.VMEM((16 * G,), jnp.float32),
        pltpu.VMEM((G,), jnp.float32),
        pltpu.SemaphoreType.DMA,
        pltpu.SemaphoreType.DMA,
    ],
)
def _seg_kernel(x_hbm, ch_hbm, b_hbm, out_hbm,
                x0_v, c0_v, b0_v, x1_v, c1_v, b1_v, acc_v, part_v,
                sem0, sem1):
    wid = lax.axis_index("s") * 2 + lax.axis_index("c")
    lane = lax.iota(jnp.int32, 16)
    lane16 = lane * 16
    zero16f = jnp.zeros((16,), jnp.float32)
    base0 = wid * ROWS

    def zero_body(j, _):
        acc_v[pl.ds(j * 16, 16)] = zero16f
        return 0

    lax.fori_loop(0, G, zero_body, 0)

    bufs = ((x0_v, c0_v, b0_v, sem0), (x1_v, c1_v, b1_v, sem1))

    def issue(cidx, buf):
        xv, cv, bv, sem = buf
        base = base0 + cidx * S
        pltpu.async_copy(x_hbm.at[pl.ds(base, S)], xv, sem)
        pltpu.async_copy(ch_hbm.at[pl.ds(base, S)], cv, sem)
        pltpu.async_copy(b_hbm.at[pl.ds(base, S)], bv, sem)

    def drain(buf):
        xv, cv, bv, sem = buf
        pltpu.make_async_copy(x_hbm.at[pl.ds(base0, S)], xv, sem).wait()
        pltpu.make_async_copy(ch_hbm.at[pl.ds(base0, S)], cv, sem).wait()
        pltpu.make_async_copy(b_hbm.at[pl.ds(base0, S)], bv, sem).wait()

    def compute(buf):
        xv, cv, bv, _ = buf

        def vec_body(i, _):
            p = xv[pl.ds(i * 16, 16)]
            c = cv[pl.ds(i * 16, 16)]
            b = bv[pl.ds(i * 16, 16)]
            plsc.addupdate_scatter(acc_v, [b * 16 + lane], p * c)
            return 0

        lax.fori_loop(0, VPC, vec_body, 0, unroll=5)

    issue(0, bufs[0])
    issue(1, bufs[1])

    def outer(o, _):
        for k in (0, 1):
            drain(bufs[k])
            compute(bufs[k])

            @pl.when(o < HALF - 1)
            def _():
                issue(2 * o + 2 + k, bufs[k])

        return 0

    lax.fori_loop(0, HALF, outer, 0)

    def fold_body(j, _):
        s = zero16f
        for m in range(16):
            km = (lane + m) & 15
            s = s + plsc.load_gather(acc_v, [j * 256 + lane16 + km])
        part_v[pl.ds(j * 16, 16)] = s
        return 0

    lax.fori_loop(0, G // 16, fold_body, 0)

    pltpu.sync_copy(part_v, out_hbm.at[wid])


def kernel(positions, node_charges, batch):
    x0 = positions[:, 0]
    c0 = node_charges[:, 0]
    partials = _seg_kernel(x0, c0, batch)
    return partials.sum(axis=0).reshape(G, 1)
